# speedup vs baseline: 4.5897x; 4.5897x over previous
import functools

import jax
import jax.numpy as jnp
from jax import lax
from jax.experimental import pallas as pl
from jax.experimental.pallas import tpu as pltpu
from jax.experimental.pallas import tpu_sc as plsc

_NC = 2
_NS = 16
_NW = _NC * _NS
_K = 128


@functools.lru_cache(maxsize=None)
def _build_sc_agg(N_pad, D, C):
    RPT = N_pad // _NS
    mesh = plsc.VectorSubcoreMesh(
        core_axis_name="c", subcore_axis_name="s",
        num_cores=_NC, num_subcores=_NS)

    def body(h_hbm, src_hbm, dst_hbm, zrows_hbm, zcnt_hbm,
             agg_out, cnt_out,
             src_v, dst_v, rows_v, ones_v, agg_sh, cnt_sh, sem):
        c = lax.axis_index("c")
        s = lax.axis_index("s")
        wid = c * _NS + s

        pltpu.sync_copy(zrows_hbm, agg_sh.at[pl.ds(s * RPT, RPT)])

        @pl.when(s == 0)
        def _():
            pltpu.sync_copy(zcnt_hbm, cnt_sh)

        pltpu.sync_copy(src_hbm.at[wid], src_v)
        pltpu.sync_copy(dst_hbm.at[wid], dst_v)

        for i in range(_K // 16):
            ones_v[pl.ds(i * 16, 16)] = jnp.ones((16,), jnp.float32)

        plsc.subcore_barrier()

        @pl.loop(0, C)
        def _(j):
            pltpu.async_copy(h_hbm.at[src_v.at[j]], rows_v, sem).wait()
            pltpu.sync_copy(rows_v, agg_sh.at[dst_v.at[j]], add=True)
            pltpu.sync_copy(ones_v, cnt_sh.at[dst_v.at[j]], add=True)

        plsc.subcore_barrier()

        pltpu.sync_copy(agg_sh.at[pl.ds(s * RPT, RPT)], agg_out.at[wid])

        @pl.when(s == 0)
        def _():
            pltpu.sync_copy(cnt_sh, cnt_out.at[c])

    return pl.kernel(
        body,
        out_type=[
            jax.ShapeDtypeStruct((_NW, RPT, D), jnp.float32),
            jax.ShapeDtypeStruct((_NC, N_pad), jnp.float32),
        ],
        mesh=mesh,
        scratch_types=[
            pltpu.VMEM((C, _K), jnp.int32),
            pltpu.VMEM((C, _K), jnp.int32),
            pltpu.VMEM((_K, D), jnp.float32),
            pltpu.VMEM((_K,), jnp.float32),
            pltpu.VMEM_SHARED((N_pad, D), jnp.float32),
            pltpu.VMEM_SHARED((N_pad,), jnp.float32),
            pltpu.SemaphoreType.DMA,
        ],
    )


def _dot(a, b):
    return jnp.dot(a, b, precision=lax.Precision.HIGHEST,
                   preferred_element_type=jnp.float32)


def _tc_layer_call(agg, cntT, h, Wl, bl, Wr, residual, head=None):
    N, D = h.shape
    BN = 1024
    grid = (pl.cdiv(N, BN),)

    def body(agg_ref, cnt_ref, h_ref, Wl_ref, bl_ref, Wr_ref, *rest):
        out_ref = rest[-1]
        cnt = cnt_ref[..., 0:1] + cnt_ref[..., 1:2]
        invc = 1.0 / jnp.maximum(cnt, 1.0)
        mean = (agg_ref[0] + agg_ref[1]) * invc
        h_blk = h_ref[...]
        y = _dot(mean, Wl_ref[...]) + bl_ref[...] + _dot(h_blk, Wr_ref[...])
        y = jnp.maximum(y, 0.0)
        if residual:
            y = y + h_blk
        if head is not None:
            Wh1_ref, bh1_ref, Wh2_ref, bh2_ref = rest[:4]
            t = jnp.maximum(_dot(y, Wh1_ref[...]) + bh1_ref[...], 0.0)
            y = _dot(t, Wh2_ref[...]) + bh2_ref[...]
        out_ref[...] = y

    w_spec = pl.BlockSpec((D, D), lambda i: (0, 0))
    b_spec = pl.BlockSpec((1, D), lambda i: (0, 0))
    in_specs = [
        pl.BlockSpec((_NC, BN, D), lambda i: (0, i, 0)),
        pl.BlockSpec((BN, _NC), lambda i: (i, 0)),
        pl.BlockSpec((BN, D), lambda i: (i, 0)),
        w_spec, b_spec, w_spec,
    ]
    args = [agg, cntT, h, Wl, bl.reshape(1, D), Wr]
    if head is not None:
        Wh1, bh1, Wh2, bh2 = head
        in_specs += [w_spec, b_spec, w_spec, b_spec]
        args += [Wh1, bh1.reshape(1, D), Wh2, bh2.reshape(1, D)]

    return pl.pallas_call(
        body,
        grid=grid,
        in_specs=in_specs,
        out_specs=pl.BlockSpec((BN, D), lambda i: (i, 0)),
        out_shape=jax.ShapeDtypeStruct((N, D), jnp.float32),
    )(*args)


def kernel(x, edge_index, Wl0, bl0, Wr0, Wl1, bl1, Wr1, Wl2, bl2, Wr2,
           Wh1, bh1, Wh2, bh2):
    N, D = x.shape
    E = edge_index.shape[1]
    C = pl.cdiv(E, _NW * _K)
    E_pad = _NW * _K * C
    N_pad = pl.cdiv(N + 1, 128) * 128
    RPT = N_pad // _NS

    src = edge_index[0]
    dst = edge_index[1]
    pad = E_pad - E
    srcp = jnp.concatenate([src, jnp.zeros((pad,), jnp.int32)]).reshape(_NW, C, _K)
    dstp = jnp.concatenate([dst, jnp.full((pad,), N, jnp.int32)]).reshape(_NW, C, _K)
    zrows = jnp.zeros((RPT, D), jnp.float32)
    zcnt = jnp.zeros((N_pad,), jnp.float32)

    sc_agg = _build_sc_agg(N_pad, D, C)
    layers = [(Wl0, bl0, Wr0), (Wl1, bl1, Wr1), (Wl2, bl2, Wr2)]

    h = x
    cntT = None
    for i, (Wl, bl, Wr) in enumerate(layers):
        agg_raw, cnt_raw = sc_agg(h, srcp, dstp, zrows, zcnt)
        agg = agg_raw.reshape(_NC, N_pad, D)
        if cntT is None:
            cntT = cnt_raw.T
        h = _tc_layer_call(agg, cntT, h, Wl, bl, Wr,
                           residual=(i > 0),
                           head=(Wh1, bh1, Wh2, bh2) if i == 2 else None)
    return h
